# baseline (speedup 1.0000x reference)
import functools

import jax
import jax.numpy as jnp
from jax import lax
from jax.experimental import pallas as pl
from jax.experimental.pallas import tpu as pltpu
from jax.experimental.pallas import tpu_sc as plsc

_ROWS_PER_BLOCK = 4096
_LANES = 16


def _sc_build_mask(mask_sites, w):
    n_sites = mask_sites.shape[0]
    mesh = plsc.VectorSubcoreMesh(core_axis_name="c", subcore_axis_name="s")

    @functools.partial(
        pl.kernel,
        mesh=mesh,
        compiler_params=pltpu.CompilerParams(needs_layout_passes=False),
        out_type=jax.ShapeDtypeStruct((w,), jnp.float32),
        scratch_types=[
            pltpu.VMEM((n_sites,), jnp.int32),
            pltpu.VMEM((w,), jnp.float32),
        ],
    )
    def build(sites_hbm, mask_hbm, idx_v, mask_v):
        is_leader = jnp.logical_and(
            lax.axis_index("c") == 0, lax.axis_index("s") == 0
        )

        @pl.when(is_leader)
        def _():
            pltpu.sync_copy(sites_hbm, idx_v)
            ones = jnp.ones((_LANES,), jnp.float32)
            for i in range(w // _LANES):
                mask_v[pl.ds(i * _LANES, _LANES)] = ones
            zeros = jnp.zeros((_LANES,), jnp.float32)
            for i in range(n_sites // _LANES):
                idx = idx_v[pl.ds(i * _LANES, _LANES)]
                plsc.store_scatter(mask_v, [idx], zeros)
            pltpu.sync_copy(mask_v, mask_hbm)

    return build(mask_sites)


def _mul_body(mask_ref, x_ref, o_ref):
    o_ref[...] = x_ref[...] * mask_ref[...]


def kernel(x, mask_sites):
    C, H, W = x.shape
    rows = C * H
    mask = _sc_build_mask(mask_sites, W).reshape(1, W)
    x2 = x.reshape(rows, W)
    n_blocks = rows // _ROWS_PER_BLOCK
    out = pl.pallas_call(
        _mul_body,
        grid=(n_blocks,),
        in_specs=[
            pl.BlockSpec((1, W), lambda i: (0, 0)),
            pl.BlockSpec((_ROWS_PER_BLOCK, W), lambda i: (i, 0)),
        ],
        out_specs=pl.BlockSpec((_ROWS_PER_BLOCK, W), lambda i: (i, 0)),
        out_shape=jax.ShapeDtypeStruct((rows, W), x.dtype),
    )(mask, x2)
    return out.reshape(C, H, W)


# speedup vs baseline: 1.1375x; 1.1375x over previous
import jax
import jax.numpy as jnp
from jax.experimental import pallas as pl
from jax.experimental.pallas import tpu as pltpu

_ROWS_PER_BLOCK = 2048


def _mask_mul_body(sites_ref, x_ref, o_ref, mask_ref):
    n_sites = sites_ref.shape[0]
    w = mask_ref.shape[1]

    @pl.when(pl.program_id(0) == 0)
    def _build_mask():
        col = jax.lax.broadcasted_iota(jnp.int32, (8, w), 1)

        def body(i, m):
            return jnp.where(col == sites_ref[i], 0.0, m)

        mask_ref[...] = jax.lax.fori_loop(
            0, n_sites, body, jnp.ones((8, w), jnp.float32)
        )

    o_ref[...] = x_ref[...] * mask_ref[0:1, :]


def kernel(x, mask_sites):
    C, H, W = x.shape
    rows = C * H
    x2 = x.reshape(rows, W)
    n_blocks = rows // _ROWS_PER_BLOCK
    out = pl.pallas_call(
        _mask_mul_body,
        grid=(n_blocks,),
        in_specs=[
            pl.BlockSpec(memory_space=pltpu.SMEM),
            pl.BlockSpec((_ROWS_PER_BLOCK, W), lambda i: (i, 0)),
        ],
        out_specs=pl.BlockSpec((_ROWS_PER_BLOCK, W), lambda i: (i, 0)),
        out_shape=jax.ShapeDtypeStruct((rows, W), x.dtype),
        scratch_shapes=[pltpu.VMEM((8, W), jnp.float32)],
    )(mask_sites, x2)
    return out.reshape(C, H, W)


# speedup vs baseline: 1.1682x; 1.0270x over previous
import jax
import jax.numpy as jnp
from jax.experimental import pallas as pl
from jax.experimental.pallas import tpu as pltpu

_ROWS_PER_BLOCK = 6144


def _mask_mul_body(sites_ref, x_ref, o_ref, mask_ref):
    n_sites = sites_ref.shape[0]
    w = mask_ref.shape[1]

    @pl.when(pl.program_id(0) == 0)
    def _build_mask():
        col = jax.lax.broadcasted_iota(jnp.int32, (8, w), 1)

        def body(i, m):
            return jnp.where(col == sites_ref[i], 0.0, m)

        mask_ref[...] = jax.lax.fori_loop(
            0, n_sites, body, jnp.ones((8, w), jnp.float32)
        )

    o_ref[...] = x_ref[...] * mask_ref[0:1, :]


def kernel(x, mask_sites):
    C, H, W = x.shape
    rows = C * H
    x2 = x.reshape(rows, W)
    n_blocks = rows // _ROWS_PER_BLOCK
    out = pl.pallas_call(
        _mask_mul_body,
        grid=(n_blocks,),
        in_specs=[
            pl.BlockSpec(memory_space=pltpu.SMEM),
            pl.BlockSpec((_ROWS_PER_BLOCK, W), lambda i: (i, 0)),
        ],
        out_specs=pl.BlockSpec((_ROWS_PER_BLOCK, W), lambda i: (i, 0)),
        out_shape=jax.ShapeDtypeStruct((rows, W), x.dtype),
        scratch_shapes=[pltpu.VMEM((8, W), jnp.float32)],
    )(mask_sites, x2)
    return out.reshape(C, H, W)
